# baseline (speedup 1.0000x reference)
import functools

import jax
import jax.numpy as jnp
from jax import lax
from jax.experimental import pallas as pl
from jax.experimental.pallas import tpu as pltpu
from jax.experimental.pallas import tpu_sc as plsc

_VOCAB = 1000
_HIDDEN = 128

_INFO = plsc.get_sparse_core_info()
_NC, _NS = _INFO.num_cores, _INFO.num_subcores
_NW = _NC * _NS


def _transform_body(e_ref, w_ref, b_ref, o_ref):
    o_ref[...] = (
        lax.dot_general(
            e_ref[...], w_ref[...],
            (((1,), (1,)), ((), ())),
            preferred_element_type=jnp.float32,
        )
        + b_ref[...]
    )


def _transform_table(embedding_table, W, b):
    return pl.pallas_call(
        _transform_body,
        out_shape=jax.ShapeDtypeStruct((_VOCAB, _HIDDEN), jnp.float32),
    )(embedding_table, W, b.reshape(1, _HIDDEN))


_NBUF = 3
_DEPTH = 2


def _make_gather(batch, seq):
    n_sh = 2
    bpw = batch // (_NW // n_sh)
    spw = seq // n_sh
    n_chunks = spw
    n_iter = -(-n_chunks // _NBUF) * _NBUF
    mesh = plsc.VectorSubcoreMesh(core_axis_name="c", subcore_axis_name="s")

    @functools.partial(
        pl.kernel,
        out_type=jax.ShapeDtypeStruct((seq, batch, _HIDDEN), jnp.float32),
        mesh=mesh,
        scratch_types=[
            pltpu.VMEM((seq, bpw), jnp.int32),
            pltpu.VMEM((_NBUF, bpw, _HIDDEN), jnp.float32),
            pltpu.VMEM_SHARED((_VOCAB, _HIDDEN), jnp.float32),
            pltpu.SemaphoreType.DMA((_NBUF,)),
            pltpu.SemaphoreType.DMA((_NBUF,)),
        ],
    )
    def gather(idx_hbm, table_hbm, out_hbm, idx_v, rows_v, table_sh,
               gsems, ssems):
        wid = lax.axis_index("s") * _NC + lax.axis_index("c")
        bg = wid // n_sh
        sh = wid % n_sh
        base = bg * bpw
        soff = sh * spw
        @pl.when(lax.axis_index("s") == 0)
        def _():
            pltpu.sync_copy(table_hbm, table_sh)

        pltpu.sync_copy(idx_hbm.at[:, pl.ds(base, bpw)], idx_v)
        plsc.subcore_barrier()

        def start_gather(j, b):
            for h in range(bpw // 128):
                pltpu.async_copy(
                    table_sh.at[idx_v.at[soff + j, pl.ds(h * 128, 128)]],
                    rows_v.at[b, pl.ds(h * 128, 128)],
                    gsems.at[b],
                )

        def wait_gather(j, b):
            for h in range(bpw // 128):
                pltpu.make_async_copy(
                    table_sh.at[idx_v.at[soff + j, pl.ds(h * 128, 128)]],
                    rows_v.at[b, pl.ds(h * 128, 128)],
                    gsems.at[b],
                ).wait()

        for j in range(_DEPTH):
            start_gather(j, j)

        def outer(g, carry):
            for b in range(_NBUF):
                k = g * _NBUF + b

                @pl.when(k < n_chunks)
                def _():
                    wait_gather(k, b)
                    pltpu.async_copy(
                        rows_v.at[b],
                        out_hbm.at[soff + k, pl.ds(base, bpw)],
                        ssems.at[b],
                    )

                b2 = (b + _DEPTH) % _NBUF
                j2 = k + _DEPTH

                @pl.when(j2 < n_chunks)
                def _():
                    @pl.when(j2 >= _NBUF)
                    def _():
                        pltpu.make_async_copy(
                            rows_v.at[b2],
                            out_hbm.at[soff, pl.ds(base, bpw)],
                            ssems.at[b2],
                        ).wait()

                    start_gather(j2, b2)
            return carry

        lax.fori_loop(0, n_iter // _NBUF, outer, 0)

        for b in range(_NBUF):
            pltpu.make_async_copy(
                rows_v.at[b], out_hbm.at[soff, pl.ds(base, bpw)],
                ssems.at[b],
            ).wait()

    return gather


_gather_4096_50 = _make_gather(4096, 50)


def kernel(input_ids, embedding_table, W, b):
    table = _transform_table(embedding_table, W, b)
    idx_t = input_ids.astype(jnp.int32).T
    out_sbh = _gather_4096_50(idx_t, table)
    return out_sbh.transpose(1, 0, 2)


# speedup vs baseline: 1.0986x; 1.0986x over previous
import functools

import jax
import jax.numpy as jnp
from jax import lax
from jax.experimental import pallas as pl
from jax.experimental.pallas import tpu as pltpu
from jax.experimental.pallas import tpu_sc as plsc

_VOCAB = 1000
_HIDDEN = 128

_INFO = plsc.get_sparse_core_info()
_NC, _NS = _INFO.num_cores, _INFO.num_subcores
_NW = _NC * _NS


def _transform_body(e_ref, w_ref, b_ref, o_ref):
    o_ref[...] = (
        lax.dot_general(
            e_ref[...], w_ref[...],
            (((1,), (1,)), ((), ())),
            preferred_element_type=jnp.float32,
        )
        + b_ref[...]
    )


def _transform_table(embedding_table, W, b):
    return pl.pallas_call(
        _transform_body,
        out_shape=jax.ShapeDtypeStruct((_VOCAB, _HIDDEN), jnp.float32),
    )(embedding_table, W, b.reshape(1, _HIDDEN))


_NBUF = 5
_DEPTH = 3


def _make_gather(batch, seq):
    bpw = batch // _NW
    n_chunks = seq
    assert n_chunks % _NBUF == 0 and n_chunks >= _NBUF
    mesh = plsc.VectorSubcoreMesh(core_axis_name="c", subcore_axis_name="s")

    @functools.partial(
        pl.kernel,
        out_type=jax.ShapeDtypeStruct((seq, batch, _HIDDEN), jnp.float32),
        mesh=mesh,
        scratch_types=[
            pltpu.VMEM((n_chunks, bpw), jnp.int32),
            pltpu.VMEM((_NBUF, bpw, _HIDDEN), jnp.float32),
            pltpu.VMEM_SHARED((_VOCAB, _HIDDEN), jnp.float32),
            pltpu.SemaphoreType.DMA((_NBUF,)),
            pltpu.SemaphoreType.DMA((_NBUF,)),
        ],
    )
    def gather(idx_hbm, table_hbm, out_hbm, idx_v, rows_v, table_sh,
               gsems, ssems):
        wid = lax.axis_index("s") * _NC + lax.axis_index("c")
        base = wid * bpw
        @pl.when(lax.axis_index("s") == 0)
        def _():
            pltpu.sync_copy(table_hbm, table_sh)

        pltpu.sync_copy(idx_hbm.at[:, pl.ds(base, bpw)], idx_v)
        plsc.subcore_barrier()

        def start_gather(j, b):
            pltpu.async_copy(
                table_sh.at[idx_v.at[j]], rows_v.at[b], gsems.at[b]
            )

        for j in range(_DEPTH):
            start_gather(j, j)

        def outer(g, carry):
            for b in range(_NBUF):
                k = g * _NBUF + b
                pltpu.make_async_copy(
                    table_sh.at[idx_v.at[k]], rows_v.at[b], gsems.at[b]
                ).wait()
                pltpu.async_copy(
                    rows_v.at[b],
                    out_hbm.at[k, pl.ds(base, bpw)],
                    ssems.at[b],
                )
                b2 = (b + _DEPTH) % _NBUF
                j2 = k + _DEPTH

                @pl.when(j2 < n_chunks)
                def _():
                    @pl.when(j2 >= _NBUF)
                    def _():
                        pltpu.make_async_copy(
                            rows_v.at[b2],
                            out_hbm.at[0, pl.ds(base, bpw)],
                            ssems.at[b2],
                        ).wait()

                    start_gather(j2, b2)
            return carry

        lax.fori_loop(0, n_chunks // _NBUF, outer, 0)

        for b in range(_NBUF):
            pltpu.make_async_copy(
                rows_v.at[b], out_hbm.at[0, pl.ds(base, bpw)], ssems.at[b]
            ).wait()

    return gather


_gather_4096_50 = _make_gather(4096, 50)


def kernel(input_ids, embedding_table, W, b):
    table = _transform_table(embedding_table, W, b)
    idx_t = input_ids.astype(jnp.int32).T
    out_sbh = _gather_4096_50(idx_t, table)
    return out_sbh.transpose(1, 0, 2)
